# baseline (speedup 1.0000x reference)
import functools

import jax
import jax.numpy as jnp
from jax import lax
from jax.experimental import pallas as pl
from jax.experimental.pallas import tpu as pltpu
from jax.experimental.pallas import tpu_sc as plsc

BATCH = 4096
SEQ = 200
HIDDEN = 64
NC = 2
NS = 16
NW = NC * NS
SEQ_PER_W = BATCH // NW

C1 = -1.5957691216057308
C2 = C1 * 0.044715


def _gelu_vec(x):
    return x / (1.0 + jnp.exp(x * (C1 + C2 * (x * x))))


def _body(seq_hbm, wt_hbm, pt_hbm, out_hbm, idx_v, rows_v, pos_v, sem):
    wid = lax.axis_index("s") * NC + lax.axis_index("c")
    pltpu.sync_copy(pt_hbm, pos_v)

    def seq_iter(it, carry):
        b = wid * SEQ_PER_W + it
        pltpu.sync_copy(seq_hbm.at[b], idx_v)
        cp0 = pltpu.async_copy(
            wt_hbm.at[idx_v.at[0]], rows_v.at[pl.ds(0, SEQ // 2)], sem)
        cp1 = pltpu.async_copy(
            wt_hbm.at[idx_v.at[1]], rows_v.at[pl.ds(SEQ // 2, SEQ // 2)], sem)
        cp0.wait()
        cp1.wait()

        def row_iter(r, carry2):
            for c in range(HIDDEN // 16):
                sl = pl.ds(c * 16, 16)
                x = rows_v[r, sl] + pos_v[r, sl]
                rows_v[r, sl] = _gelu_vec(x)
            return carry2

        lax.fori_loop(0, SEQ, row_iter, 0)
        pltpu.sync_copy(rows_v, out_hbm.at[b])
        return carry

    lax.fori_loop(0, SEQ_PER_W, seq_iter, 0)


def kernel(input_seq, word_table, pos_table):
    seq3 = input_seq.astype(jnp.int32).reshape(BATCH, 2, SEQ // 2)
    mesh = plsc.VectorSubcoreMesh(core_axis_name="c", subcore_axis_name="s")
    run = functools.partial(
        pl.kernel,
        mesh=mesh,
        out_type=jax.ShapeDtypeStruct((BATCH, SEQ, HIDDEN), jnp.float32),
        compiler_params=pltpu.CompilerParams(use_tc_tiling_on_sc=False),
        scratch_types=[
            pltpu.VMEM((2, SEQ // 2), jnp.int32),
            pltpu.VMEM((SEQ, HIDDEN), jnp.float32),
            pltpu.VMEM((SEQ, HIDDEN), jnp.float32),
            pltpu.SemaphoreType.DMA,
        ],
    )(_body)
    return run(seq3, word_table, pos_table)


# speedup vs baseline: 1.3664x; 1.3664x over previous
import functools

import jax
import jax.numpy as jnp
from jax import lax
from jax.experimental import pallas as pl
from jax.experimental.pallas import tpu as pltpu
from jax.experimental.pallas import tpu_sc as plsc

BATCH = 4096
SEQ = 200
HIDDEN = 64
NC = 2
NS = 16
NW = NC * NS
SEQ_PER_W = BATCH // NW
NSTEPS = SEQ_PER_W // 2
HALF = SEQ // 2

C1 = -1.5957691216057308
C2 = C1 * 0.044715


def _gelu_vec(x):
    return x / (1.0 + jnp.exp(x * (C1 + C2 * (x * x))))


def _body(seq_hbm, wt_hbm, pt_hbm, out_hbm, idx_v, rows_v, pos_v,
          gs0, gs1, ws0, ws1):
    wid = lax.axis_index("s") * NC + lax.axis_index("c")
    pltpu.sync_copy(pt_hbm, pos_v)
    pltpu.sync_copy(seq_hbm.at[wid], idx_v)

    def issue_gather(g, buf, sem):
        for j in range(2):
            pltpu.async_copy(
                wt_hbm.at[idx_v.at[g, j]],
                rows_v.at[buf, pl.ds(j * HALF, HALF)],
                sem)

    def wait_gather(buf, sem):
        for j in range(2):
            pltpu.make_async_copy(
                wt_hbm.at[idx_v.at[0, j]],
                rows_v.at[buf, pl.ds(j * HALF, HALF)],
                sem).wait()

    def issue_wb(g, buf, sem):
        pltpu.async_copy(rows_v.at[buf], out_hbm.at[wid * SEQ_PER_W + g], sem)

    def wait_wb(buf, sem):
        pltpu.make_async_copy(rows_v.at[buf], out_hbm.at[0], sem).wait()

    def compute(buf):
        def body(i, c):
            for rr in range(4):
                r = i * 4 + rr
                for cc in range(HIDDEN // 16):
                    sl = pl.ds(cc * 16, 16)
                    x = rows_v[buf, r, sl] + pos_v[r, sl]
                    rows_v[buf, r, sl] = _gelu_vec(x)
            return c
        lax.fori_loop(0, SEQ // 4, body, 0)

    issue_gather(0, 0, gs0)

    def step(s, carry):
        g0 = 2 * s

        @pl.when(s > 0)
        def _():
            wait_wb(1, ws1)

        issue_gather(g0 + 1, 1, gs1)
        wait_gather(0, gs0)
        compute(0)
        issue_wb(g0, 0, ws0)
        wait_gather(1, gs1)
        compute(1)
        issue_wb(g0 + 1, 1, ws1)
        wait_wb(0, ws0)

        @pl.when(s < NSTEPS - 1)
        def _():
            issue_gather(g0 + 2, 0, gs0)

        return carry

    lax.fori_loop(0, NSTEPS, step, 0)
    wait_wb(1, ws1)


def kernel(input_seq, word_table, pos_table):
    seq4 = input_seq.astype(jnp.int32).reshape(NW, SEQ_PER_W, 2, HALF)
    mesh = plsc.VectorSubcoreMesh(core_axis_name="c", subcore_axis_name="s")
    run = functools.partial(
        pl.kernel,
        mesh=mesh,
        out_type=jax.ShapeDtypeStruct((BATCH, SEQ, HIDDEN), jnp.float32),
        compiler_params=pltpu.CompilerParams(use_tc_tiling_on_sc=False),
        scratch_types=[
            pltpu.VMEM((SEQ_PER_W, 2, HALF), jnp.int32),
            pltpu.VMEM((2, SEQ, HIDDEN), jnp.float32),
            pltpu.VMEM((SEQ, HIDDEN), jnp.float32),
            pltpu.SemaphoreType.DMA,
            pltpu.SemaphoreType.DMA,
            pltpu.SemaphoreType.DMA,
            pltpu.SemaphoreType.DMA,
        ],
    )(_body)
    return run(seq4, word_table, pos_table)
